# Initial kernel scaffold; baseline (speedup 1.0000x reference)
import jax
import jax.numpy as jnp
from jax.experimental import pallas as pl


def kernel(h, edge_index, edge_weight, W):
    raise NotImplementedError("write your pallas kernel here")


# speedup vs baseline: 6.2806x; 6.2806x over previous
import functools

import jax
import jax.numpy as jnp
from jax import lax
from jax.experimental import pallas as pl
from jax.experimental.pallas import tpu as pltpu
from jax.experimental.pallas import tpu_sc as plsc

NC = 2
NS = 16
LANES = 16
B = 80
ZR = 16


def _spmm_body(n_rows, n_batches,
               h_hbm, src_hbm, dst_hbm, w_hbm, out_hbm,
               acc, idx_v, dst_v, w_v, rows_v, zbuf, sem):
  c = lax.axis_index("c")
  s = lax.axis_index("s")
  wid = c * NS + s

  base_rows = (n_rows // (NS * ZR)) * ZR
  row_base = s * base_rows
  tail = n_rows - (NS - 1) * base_rows - base_rows

  zero = jnp.zeros((LANES,), jnp.float32)

  def zrow(i, carry):
    for j in range(128 // LANES):
      zbuf[i, pl.ds(j * LANES, LANES)] = zero
    return carry

  lax.fori_loop(0, ZR, zrow, 0)

  def zchunk(k, carry):
    pltpu.sync_copy(zbuf, acc.at[pl.ds(row_base + k * ZR, ZR)])
    return carry

  lax.fori_loop(0, base_rows // ZR, zchunk, 0)

  @pl.when(s == NS - 1)
  def _zero_tail():
    pltpu.sync_copy(zbuf, acc.at[pl.ds(n_rows - tail, tail)])

  ep = n_batches * B
  pltpu.sync_copy(src_hbm.at[pl.ds(wid * ep, ep)], idx_v)
  pltpu.sync_copy(dst_hbm.at[wid], dst_v)
  pltpu.sync_copy(w_hbm.at[pl.ds(wid * ep, ep)], w_v)

  plsc.subcore_barrier()

  def batch(t, carry):
    pltpu.async_copy(h_hbm.at[idx_v.at[pl.ds(t * B, B)]], rows_v, sem).wait()

    def scale(g, carry2):
      wvec = w_v[pl.ds(t * B + g * LANES, LANES)]
      base = g * LANES
      for l in range(LANES):
        w = wvec[l]
        i = base + l
        for j in range(128 // LANES):
          sl = pl.ds(j * LANES, LANES)
          rows_v[i, sl] = rows_v[i, sl] * w
      return carry2

    lax.fori_loop(0, B // LANES, scale, 0)

    pltpu.sync_copy(rows_v, acc.at[dst_v.at[t]], add=True)
    return carry

  lax.fori_loop(0, n_batches, batch, 0)

  plsc.subcore_barrier()

  pltpu.sync_copy(acc.at[pl.ds(row_base, base_rows)],
                  out_hbm.at[c, pl.ds(row_base, base_rows)])

  @pl.when(s == NS - 1)
  def _dump_tail():
    pltpu.sync_copy(acc.at[pl.ds(n_rows - tail, tail)],
                    out_hbm.at[c, pl.ds(n_rows - tail, tail)])


def _linear_body(p_ref, w_ref, o_ref):
  y = p_ref[0] + p_ref[1]
  o_ref[...] = lax.dot_general(y, w_ref[...], (((1,), (1,)), ((), ())),
                               preferred_element_type=jnp.float32)


def kernel(h, edge_index, edge_weight, W):
  n, d = h.shape
  e = edge_weight.shape[0]
  nw = NC * NS
  assert e % (nw * B) == 0 and d == 128
  n_batches = e // (nw * B)
  assert B % LANES == 0

  ep = n_batches * B
  src1 = edge_index[1]
  dst3 = edge_index[0].reshape(nw, n_batches, B)

  mesh = plsc.VectorSubcoreMesh(core_axis_name="c", subcore_axis_name="s")
  spmm = pl.kernel(
      functools.partial(_spmm_body, n, n_batches),
      out_type=jax.ShapeDtypeStruct((NC, n, d), jnp.float32),
      mesh=mesh,
      scratch_types=[
          pltpu.VMEM_SHARED((n, d), jnp.float32),
          pltpu.VMEM((ep,), jnp.int32),
          pltpu.VMEM((n_batches, B), jnp.int32),
          pltpu.VMEM((ep,), jnp.float32),
          pltpu.VMEM((B, d), jnp.float32),
          pltpu.VMEM((ZR, d), jnp.float32),
          pltpu.SemaphoreType.DMA,
      ],
  )
  partials = spmm(h, src1, dst3, edge_weight)

  res = pl.pallas_call(
      _linear_body,
      out_shape=jax.ShapeDtypeStruct((n, d), jnp.float32),
  )(partials, W)
  return res
